# Initial kernel scaffold; baseline (speedup 1.0000x reference)
import jax
import jax.numpy as jnp
from jax.experimental import pallas as pl


def kernel(x, router_logits, w1, w3, w2):
    raise NotImplementedError("write your pallas kernel here")


# speedup vs baseline: 1.4704x; 1.4704x over previous
import functools

import jax
import jax.numpy as jnp
from jax import lax
from jax.experimental import pallas as pl
from jax.experimental.pallas import tpu as pltpu
from jax.experimental.pallas import tpu_sc as plsc

T = 2048
D = 2048
F = 2048
E = 8
K = 2

BR = 128
NBLK = T * K // BR + E - 1
NSLOT = NBLK * BR

NC = 2
NS = 16
NW = NC * NS
CHUNK = 8
ROWS_PER_W = (T * K) // NW
NCHUNK = ROWS_PER_W // CHUNK


def _route_body(logits_ref, tw_ref, slots_ref, meta_ref):
    logits = logits_ref[...]
    m = jnp.max(logits, axis=-1, keepdims=True)
    ex = jnp.exp(logits - m)
    p = ex / jnp.sum(ex, axis=-1, keepdims=True)
    e_ids = lax.broadcasted_iota(jnp.int32, (T, E), 1)
    w0 = jnp.max(p, axis=-1, keepdims=True)
    i0 = jnp.min(jnp.where(p == w0, e_ids, E), axis=-1, keepdims=True)
    p2 = jnp.where(e_ids == i0, -jnp.inf, p)
    w1_ = jnp.max(p2, axis=-1, keepdims=True)
    i1 = jnp.min(jnp.where(p2 == w1_, e_ids, E), axis=-1, keepdims=True)
    sel0 = e_ids == i0
    sel1 = e_ids == i1
    cnt = (sel0 | sel1).astype(jnp.float32)
    ri = lax.broadcasted_iota(jnp.int32, (T, 1), 0)
    ci = lax.broadcasted_iota(jnp.int32, (1, T), 1)
    lt = (ci < ri).astype(jnp.bfloat16)
    rank = jnp.dot(lt, cnt.astype(jnp.bfloat16), preferred_element_type=jnp.float32)
    totals = jnp.sum(cnt, axis=0, keepdims=True)
    nblk = jnp.ceil(totals / BR)
    er = lax.broadcasted_iota(jnp.int32, (E, 1), 0)
    ec = lax.broadcasted_iota(jnp.int32, (1, E), 1)
    me = (er < ec).astype(jnp.float32)
    blk_off = jnp.dot(nblk, me, preferred_element_type=jnp.float32)
    slot = blk_off * BR + rank
    slot0 = jnp.sum(jnp.where(sel0, slot, 0.0), axis=-1, keepdims=True)
    slot1 = jnp.sum(jnp.where(sel1, slot, 0.0), axis=-1, keepdims=True)
    tw_ref[...] = jnp.concatenate([w0, w1_], axis=1)
    slots_ref[...] = jnp.concatenate([slot0, slot1], axis=1).astype(jnp.int32)
    b_ids = lax.broadcasted_iota(jnp.int32, (1, NBLK), 1)
    nb_i = jnp.sum(nblk).astype(jnp.int32)
    b_eff = jnp.minimum(b_ids, nb_i - 1).astype(jnp.float32)
    cmp = (b_eff >= jnp.reshape(blk_off, (E, 1))).astype(jnp.int32)
    be = jnp.sum(cmp, axis=0, keepdims=True) - 1
    meta_ref[...] = jnp.concatenate([be, jnp.reshape(nb_i, (1, 1))], axis=1)


def _route(router_logits):
    return pl.pallas_call(
        _route_body,
        out_shape=[
            jax.ShapeDtypeStruct((T, K), jnp.float32),
            jax.ShapeDtypeStruct((T, K), jnp.int32),
            jax.ShapeDtypeStruct((1, NBLK + 1), jnp.int32),
        ],
    )(router_logits)


def _sc_scatter(x, idx3):
    mesh = plsc.VectorSubcoreMesh(core_axis_name="c", subcore_axis_name="s")

    @functools.partial(
        pl.kernel,
        mesh=mesh,
        out_type=jax.ShapeDtypeStruct((NSLOT, D), jnp.float32),
        scratch_types=[
            pltpu.VMEM((CHUNK,), jnp.int32),
            pltpu.VMEM((CHUNK, D), jnp.float32),
            pltpu.SemaphoreType.DMA,
        ],
    )
    def k(x_hbm, idx_hbm, xs_hbm, idx_v, rows_v, sem):
        wid = lax.axis_index("s") * NC + lax.axis_index("c")
        tok0 = (wid % (T // ROWS_PER_W)) * ROWS_PER_W

        @pl.loop(0, NCHUNK)
        def _(c):
            pltpu.sync_copy(idx_hbm.at[wid, c], idx_v)
            pltpu.sync_copy(x_hbm.at[pl.ds(tok0 + c * CHUNK, CHUNK)], rows_v)
            pltpu.async_copy(rows_v, xs_hbm.at[idx_v], sem).wait()

    return k(x, idx3)


def _sc_gather(y, idx3):
    mesh = plsc.VectorSubcoreMesh(core_axis_name="c", subcore_axis_name="s")

    @functools.partial(
        pl.kernel,
        mesh=mesh,
        out_type=jax.ShapeDtypeStruct((T * K, D), jnp.float32),
        scratch_types=[
            pltpu.VMEM((CHUNK,), jnp.int32),
            pltpu.VMEM((CHUNK, D), jnp.float32),
            pltpu.SemaphoreType.DMA,
        ],
    )
    def k(y_hbm, idx_hbm, yg_hbm, idx_v, rows_v, sem):
        wid = lax.axis_index("s") * NC + lax.axis_index("c")
        base = wid * ROWS_PER_W

        @pl.loop(0, NCHUNK)
        def _(c):
            pltpu.sync_copy(idx_hbm.at[wid, c], idx_v)
            pltpu.async_copy(y_hbm.at[idx_v], rows_v, sem).wait()
            pltpu.sync_copy(rows_v, yg_hbm.at[pl.ds(base + c * CHUNK, CHUNK)])

    return k(y, idx3)


def _ffn_body(meta_ref, xs_ref, w1_ref, w3_ref, w2_ref, y_ref):
    b = pl.program_id(0)
    nvalid = meta_ref[NBLK]

    @pl.when(b < nvalid)
    def _():
        xb = xs_ref[...].astype(jnp.bfloat16)
        gate = jnp.dot(xb, w1_ref[0], preferred_element_type=jnp.float32)
        up = jnp.dot(xb, w3_ref[0], preferred_element_type=jnp.float32)
        h = (gate * lax.logistic(gate) * up).astype(jnp.bfloat16)
        y_ref[...] = jnp.dot(h, w2_ref[0], preferred_element_type=jnp.float32)


def _ffn(meta, xs, w1b, w3b, w2b):
    grid_spec = pltpu.PrefetchScalarGridSpec(
        num_scalar_prefetch=1,
        grid=(NBLK,),
        in_specs=[
            pl.BlockSpec((BR, D), lambda b, m: (b, 0)),
            pl.BlockSpec((1, D, F), lambda b, m: (m[b], 0, 0)),
            pl.BlockSpec((1, D, F), lambda b, m: (m[b], 0, 0)),
            pl.BlockSpec((1, F, D), lambda b, m: (m[b], 0, 0)),
        ],
        out_specs=pl.BlockSpec((BR, D), lambda b, m: (b, 0)),
    )
    return pl.pallas_call(
        _ffn_body,
        grid_spec=grid_spec,
        out_shape=jax.ShapeDtypeStruct((NSLOT, D), jnp.float32),
        compiler_params=pltpu.CompilerParams(
            dimension_semantics=("arbitrary",),
        ),
    )(meta, xs, w1b, w3b, w2b)


def _combine_body(y0_ref, y1_ref, tw_ref, o_ref):
    tw = tw_ref[...]
    o_ref[...] = tw[:, 0:1] * y0_ref[...] + tw[:, 1:2] * y1_ref[...]


def _combine(yg, tw):
    nb = T // BR
    return pl.pallas_call(
        _combine_body,
        grid=(nb,),
        in_specs=[
            pl.BlockSpec((BR, D), lambda i: (i, 0)),
            pl.BlockSpec((BR, D), lambda i: (i + nb, 0)),
            pl.BlockSpec((BR, K), lambda i: (i, 0)),
        ],
        out_specs=pl.BlockSpec((BR, D), lambda i: (i, 0)),
        out_shape=jax.ShapeDtypeStruct((T, D), jnp.float32),
    )(yg, yg, tw)


@jax.jit
def kernel(x, router_logits, w1, w3, w2):
    tw, slots, meta = _route(router_logits)
    slots_flat = jnp.concatenate([slots[:, 0], slots[:, 1]])
    idx3 = jnp.reshape(slots_flat, (NW, NCHUNK, CHUNK))
    xs = _sc_scatter(x, idx3)
    w1b = w1.astype(jnp.bfloat16)
    w3b = w3.astype(jnp.bfloat16)
    w2b = w2.astype(jnp.bfloat16)
    y = _ffn(jnp.reshape(meta, (NBLK + 1,)), xs, w1b, w3b, w2b)
    yg = _sc_gather(y, idx3)
    return _combine(yg, tw)


# speedup vs baseline: 1.5233x; 1.0359x over previous
import functools

import jax
import jax.numpy as jnp
from jax import lax
from jax.experimental import pallas as pl
from jax.experimental.pallas import tpu as pltpu
from jax.experimental.pallas import tpu_sc as plsc

T = 2048
D = 2048
F = 2048
E = 8
K = 2

BR = 128
NBLK = T * K // BR + E - 1
NSLOT = NBLK * BR

NC = 2
NS = 16
NW = NC * NS
CHUNK = 16
ROWS_PER_W = (T * K) // NW
NCHUNK = ROWS_PER_W // CHUNK


def _route_body(logits_ref, tw_ref, slots_ref, meta_ref):
    logits = logits_ref[...]
    m = jnp.max(logits, axis=-1, keepdims=True)
    ex = jnp.exp(logits - m)
    p = ex / jnp.sum(ex, axis=-1, keepdims=True)
    e_ids = lax.broadcasted_iota(jnp.int32, (T, E), 1)
    w0 = jnp.max(p, axis=-1, keepdims=True)
    i0 = jnp.min(jnp.where(p == w0, e_ids, E), axis=-1, keepdims=True)
    p2 = jnp.where(e_ids == i0, -jnp.inf, p)
    w1_ = jnp.max(p2, axis=-1, keepdims=True)
    i1 = jnp.min(jnp.where(p2 == w1_, e_ids, E), axis=-1, keepdims=True)
    sel0 = e_ids == i0
    sel1 = e_ids == i1
    cnt = (sel0 | sel1).astype(jnp.float32)
    ri = lax.broadcasted_iota(jnp.int32, (T, 1), 0)
    ci = lax.broadcasted_iota(jnp.int32, (1, T), 1)
    lt = (ci < ri).astype(jnp.bfloat16)
    rank = jnp.dot(lt, cnt.astype(jnp.bfloat16), preferred_element_type=jnp.float32)
    totals = jnp.sum(cnt, axis=0, keepdims=True)
    nblk = jnp.ceil(totals / BR)
    er = lax.broadcasted_iota(jnp.int32, (E, 1), 0)
    ec = lax.broadcasted_iota(jnp.int32, (1, E), 1)
    me = (er < ec).astype(jnp.float32)
    blk_off = jnp.dot(nblk, me, preferred_element_type=jnp.float32)
    slot = blk_off * BR + rank
    slot0 = jnp.sum(jnp.where(sel0, slot, 0.0), axis=-1, keepdims=True)
    slot1 = jnp.sum(jnp.where(sel1, slot, 0.0), axis=-1, keepdims=True)
    tw_ref[...] = jnp.concatenate([w0, w1_], axis=1)
    slots_ref[...] = jnp.concatenate([slot0, slot1], axis=1).astype(jnp.int32)
    b_ids = lax.broadcasted_iota(jnp.int32, (1, NBLK), 1)
    nb_i = jnp.sum(nblk).astype(jnp.int32)
    b_eff = jnp.minimum(b_ids, nb_i - 1).astype(jnp.float32)
    cmp = (b_eff >= jnp.reshape(blk_off, (E, 1))).astype(jnp.int32)
    be = jnp.sum(cmp, axis=0, keepdims=True) - 1
    meta_ref[...] = jnp.concatenate([be, jnp.reshape(nb_i, (1, 1))], axis=1)


def _route(router_logits):
    return pl.pallas_call(
        _route_body,
        out_shape=[
            jax.ShapeDtypeStruct((T, K), jnp.float32),
            jax.ShapeDtypeStruct((T, K), jnp.int32),
            jax.ShapeDtypeStruct((1, NBLK + 1), jnp.int32),
        ],
    )(router_logits)


def _sc_scatter(x, idx3):
    mesh = plsc.VectorSubcoreMesh(core_axis_name="c", subcore_axis_name="s")

    @functools.partial(
        pl.kernel,
        mesh=mesh,
        out_type=jax.ShapeDtypeStruct((NSLOT, D), jnp.float32),
        scratch_types=[
            pltpu.VMEM((NCHUNK, CHUNK), jnp.int32),
            pltpu.VMEM((CHUNK, D), jnp.float32),
            pltpu.VMEM((CHUNK, D), jnp.float32),
            pltpu.SemaphoreType.DMA,
            pltpu.SemaphoreType.DMA,
        ],
    )
    def k(x_hbm, idx_hbm, xs_hbm, idx_v, rows_a, rows_b, sem_a, sem_b):
        wid = lax.axis_index("s") * NC + lax.axis_index("c")
        tok0 = (wid % (T // ROWS_PER_W)) * ROWS_PER_W
        pltpu.sync_copy(idx_hbm.at[wid], idx_v)
        bufs = (rows_a, rows_b)
        sems = (sem_a, sem_b)
        for b in range(2):
            pltpu.async_copy(
                x_hbm.at[pl.ds(tok0 + b * CHUNK, CHUNK)], bufs[b], sems[b]
            )

        @pl.loop(0, NCHUNK // 2)
        def _(i):
            for b in range(2):
                cc = i * 2 + b
                pltpu.make_async_copy(
                    x_hbm.at[pl.ds(tok0, CHUNK)], bufs[b], sems[b]
                ).wait()
                pltpu.sync_copy(bufs[b], xs_hbm.at[idx_v.at[cc]])

                @pl.when(cc + 2 < NCHUNK)
                def _():
                    pltpu.async_copy(
                        x_hbm.at[pl.ds(tok0 + (cc + 2) * CHUNK, CHUNK)],
                        bufs[b],
                        sems[b],
                    )

    return k(x, idx3)


def _sc_gather(y, idx3):
    mesh = plsc.VectorSubcoreMesh(core_axis_name="c", subcore_axis_name="s")

    @functools.partial(
        pl.kernel,
        mesh=mesh,
        out_type=jax.ShapeDtypeStruct((T * K, D), jnp.float32),
        scratch_types=[
            pltpu.VMEM((NCHUNK, CHUNK), jnp.int32),
            pltpu.VMEM((CHUNK, D), jnp.float32),
            pltpu.VMEM((CHUNK, D), jnp.float32),
            pltpu.SemaphoreType.DMA,
            pltpu.SemaphoreType.DMA,
        ],
    )
    def k(y_hbm, idx_hbm, yg_hbm, idx_v, rows_a, rows_b, sem_a, sem_b):
        wid = lax.axis_index("s") * NC + lax.axis_index("c")
        base = wid * ROWS_PER_W
        pltpu.sync_copy(idx_hbm.at[wid], idx_v)
        bufs = (rows_a, rows_b)
        sems = (sem_a, sem_b)
        for b in range(2):
            pltpu.async_copy(y_hbm.at[idx_v.at[b]], bufs[b], sems[b])

        @pl.loop(0, NCHUNK // 2)
        def _(i):
            for b in range(2):
                cc = i * 2 + b
                pltpu.make_async_copy(
                    y_hbm.at[idx_v.at[0]], bufs[b], sems[b]
                ).wait()
                pltpu.sync_copy(
                    bufs[b], yg_hbm.at[pl.ds(base + cc * CHUNK, CHUNK)]
                )

                @pl.when(cc + 2 < NCHUNK)
                def _():
                    pltpu.async_copy(y_hbm.at[idx_v.at[cc + 2]], bufs[b], sems[b])

    return k(y, idx3)


def _ffn_body(meta_ref, xs_ref, w1_ref, w3_ref, w2_ref, y_ref):
    b = pl.program_id(0)
    nvalid = meta_ref[NBLK]

    @pl.when(b < nvalid)
    def _():
        xb = xs_ref[...].astype(jnp.bfloat16)
        gate = jnp.dot(xb, w1_ref[0], preferred_element_type=jnp.float32)
        up = jnp.dot(xb, w3_ref[0], preferred_element_type=jnp.float32)
        h = (gate * lax.logistic(gate) * up).astype(jnp.bfloat16)
        y_ref[...] = jnp.dot(h, w2_ref[0], preferred_element_type=jnp.float32)


def _ffn(meta, xs, w1b, w3b, w2b):
    grid_spec = pltpu.PrefetchScalarGridSpec(
        num_scalar_prefetch=1,
        grid=(NBLK,),
        in_specs=[
            pl.BlockSpec((BR, D), lambda b, m: (b, 0)),
            pl.BlockSpec((1, D, F), lambda b, m: (m[b], 0, 0)),
            pl.BlockSpec((1, D, F), lambda b, m: (m[b], 0, 0)),
            pl.BlockSpec((1, F, D), lambda b, m: (m[b], 0, 0)),
        ],
        out_specs=pl.BlockSpec((BR, D), lambda b, m: (b, 0)),
    )
    return pl.pallas_call(
        _ffn_body,
        grid_spec=grid_spec,
        out_shape=jax.ShapeDtypeStruct((NSLOT, D), jnp.float32),
        compiler_params=pltpu.CompilerParams(
            dimension_semantics=("parallel",),
        ),
    )(meta, xs, w1b, w3b, w2b)


def _combine_body(y0_ref, y1_ref, tw_ref, o_ref):
    tw = tw_ref[...]
    o_ref[...] = tw[:, 0:1] * y0_ref[...] + tw[:, 1:2] * y1_ref[...]


def _combine(yg, tw):
    nb = T // BR
    return pl.pallas_call(
        _combine_body,
        grid=(nb,),
        in_specs=[
            pl.BlockSpec((BR, D), lambda i: (i, 0)),
            pl.BlockSpec((BR, D), lambda i: (i + nb, 0)),
            pl.BlockSpec((BR, K), lambda i: (i, 0)),
        ],
        out_specs=pl.BlockSpec((BR, D), lambda i: (i, 0)),
        out_shape=jax.ShapeDtypeStruct((T, D), jnp.float32),
        compiler_params=pltpu.CompilerParams(
            dimension_semantics=("parallel",),
        ),
    )(yg, yg, tw)


@jax.jit
def kernel(x, router_logits, w1, w3, w2):
    tw, slots, meta = _route(router_logits)
    slots_flat = jnp.concatenate([slots[:, 0], slots[:, 1]])
    idx3 = jnp.reshape(slots_flat, (NW, NCHUNK, CHUNK))
    xs = _sc_scatter(x, idx3)
    w1b = w1.astype(jnp.bfloat16)
    w3b = w3.astype(jnp.bfloat16)
    w2b = w2.astype(jnp.bfloat16)
    y = _ffn(jnp.reshape(meta, (NBLK + 1,)), xs, w1b, w3b, w2b)
    yg = _sc_gather(y, idx3)
    return _combine(yg, tw)


# speedup vs baseline: 1.7429x; 1.1442x over previous
import functools

import jax
import jax.numpy as jnp
from jax import lax
from jax.experimental import pallas as pl
from jax.experimental.pallas import tpu as pltpu
from jax.experimental.pallas import tpu_sc as plsc

T = 2048
D = 2048
F = 2048
E = 8
K = 2

BR = 256
NBLK = T * K // BR + E - 1
NSLOT = NBLK * BR

NC = 2
NS = 16
NW = NC * NS
CHUNK = 16
ROWS_PER_W = (T * K) // NW
NCHUNK = ROWS_PER_W // CHUNK


def _route_body(logits_ref, tw_ref, slots_ref, meta_ref):
    logits = logits_ref[...]
    m = jnp.max(logits, axis=-1, keepdims=True)
    ex = jnp.exp(logits - m)
    p = ex / jnp.sum(ex, axis=-1, keepdims=True)
    e_ids = lax.broadcasted_iota(jnp.int32, (T, E), 1)
    w0 = jnp.max(p, axis=-1, keepdims=True)
    i0 = jnp.min(jnp.where(p == w0, e_ids, E), axis=-1, keepdims=True)
    p2 = jnp.where(e_ids == i0, -jnp.inf, p)
    w1_ = jnp.max(p2, axis=-1, keepdims=True)
    i1 = jnp.min(jnp.where(p2 == w1_, e_ids, E), axis=-1, keepdims=True)
    sel0 = e_ids == i0
    sel1 = e_ids == i1
    cnt = (sel0 | sel1).astype(jnp.float32)
    ri = lax.broadcasted_iota(jnp.int32, (T, 1), 0)
    ci = lax.broadcasted_iota(jnp.int32, (1, T), 1)
    lt = (ci < ri).astype(jnp.bfloat16)
    rank = jnp.dot(lt, cnt.astype(jnp.bfloat16), preferred_element_type=jnp.float32)
    totals = jnp.sum(cnt, axis=0, keepdims=True)
    nblk = jnp.ceil(totals / BR)
    er = lax.broadcasted_iota(jnp.int32, (E, 1), 0)
    ec = lax.broadcasted_iota(jnp.int32, (1, E), 1)
    me = (er < ec).astype(jnp.float32)
    blk_off = jnp.dot(nblk, me, preferred_element_type=jnp.float32)
    slot = blk_off * BR + rank
    slot0 = jnp.sum(jnp.where(sel0, slot, 0.0), axis=-1, keepdims=True)
    slot1 = jnp.sum(jnp.where(sel1, slot, 0.0), axis=-1, keepdims=True)
    tw_ref[...] = jnp.concatenate([w0, w1_], axis=1)
    slots_ref[...] = jnp.concatenate([slot0, slot1], axis=1).astype(jnp.int32)
    b_ids = lax.broadcasted_iota(jnp.int32, (1, NBLK), 1)
    nb_i = jnp.sum(nblk).astype(jnp.int32)
    b_eff = jnp.minimum(b_ids, nb_i - 1).astype(jnp.float32)
    cmp = (b_eff >= jnp.reshape(blk_off, (E, 1))).astype(jnp.int32)
    be = jnp.sum(cmp, axis=0, keepdims=True) - 1
    meta_ref[...] = jnp.concatenate([be, jnp.reshape(nb_i, (1, 1))], axis=1)


def _route(router_logits):
    return pl.pallas_call(
        _route_body,
        out_shape=[
            jax.ShapeDtypeStruct((T, K), jnp.float32),
            jax.ShapeDtypeStruct((T, K), jnp.int32),
            jax.ShapeDtypeStruct((1, NBLK + 1), jnp.int32),
        ],
    )(router_logits)


def _sc_scatter(x, idx3):
    mesh = plsc.VectorSubcoreMesh(core_axis_name="c", subcore_axis_name="s")

    @functools.partial(
        pl.kernel,
        mesh=mesh,
        out_type=jax.ShapeDtypeStruct((NSLOT, D), jnp.float32),
        scratch_types=[
            pltpu.VMEM((NCHUNK, CHUNK), jnp.int32),
            pltpu.VMEM((CHUNK, D), jnp.float32),
            pltpu.VMEM((CHUNK, D), jnp.float32),
            pltpu.SemaphoreType.DMA,
            pltpu.SemaphoreType.DMA,
        ],
    )
    def k(x_hbm, idx_hbm, xs_hbm, idx_v, rows_a, rows_b, sem_a, sem_b):
        wid = lax.axis_index("s") * NC + lax.axis_index("c")
        tok0 = (wid % (T // ROWS_PER_W)) * ROWS_PER_W
        pltpu.sync_copy(idx_hbm.at[wid], idx_v)
        bufs = (rows_a, rows_b)
        sems = (sem_a, sem_b)
        for b in range(2):
            pltpu.async_copy(
                x_hbm.at[pl.ds(tok0 + b * CHUNK, CHUNK)], bufs[b], sems[b]
            )

        @pl.loop(0, NCHUNK // 2)
        def _(i):
            for b in range(2):
                cc = i * 2 + b
                pltpu.make_async_copy(
                    x_hbm.at[pl.ds(tok0, CHUNK)], bufs[b], sems[b]
                ).wait()
                pltpu.sync_copy(bufs[b], xs_hbm.at[idx_v.at[cc]])

                @pl.when(cc + 2 < NCHUNK)
                def _():
                    pltpu.async_copy(
                        x_hbm.at[pl.ds(tok0 + (cc + 2) * CHUNK, CHUNK)],
                        bufs[b],
                        sems[b],
                    )

    return k(x, idx3)


def _sc_gather(y, idx3):
    mesh = plsc.VectorSubcoreMesh(core_axis_name="c", subcore_axis_name="s")

    @functools.partial(
        pl.kernel,
        mesh=mesh,
        out_type=jax.ShapeDtypeStruct((T * K, D), jnp.float32),
        scratch_types=[
            pltpu.VMEM((NCHUNK, CHUNK), jnp.int32),
            pltpu.VMEM((CHUNK, D), jnp.float32),
            pltpu.VMEM((CHUNK, D), jnp.float32),
            pltpu.SemaphoreType.DMA,
            pltpu.SemaphoreType.DMA,
        ],
    )
    def k(y_hbm, idx_hbm, yg_hbm, idx_v, rows_a, rows_b, sem_a, sem_b):
        wid = lax.axis_index("s") * NC + lax.axis_index("c")
        base = wid * ROWS_PER_W
        pltpu.sync_copy(idx_hbm.at[wid], idx_v)
        bufs = (rows_a, rows_b)
        sems = (sem_a, sem_b)
        for b in range(2):
            pltpu.async_copy(y_hbm.at[idx_v.at[b]], bufs[b], sems[b])

        @pl.loop(0, NCHUNK // 2)
        def _(i):
            for b in range(2):
                cc = i * 2 + b
                pltpu.make_async_copy(
                    y_hbm.at[idx_v.at[0]], bufs[b], sems[b]
                ).wait()
                pltpu.sync_copy(
                    bufs[b], yg_hbm.at[pl.ds(base + cc * CHUNK, CHUNK)]
                )

                @pl.when(cc + 2 < NCHUNK)
                def _():
                    pltpu.async_copy(y_hbm.at[idx_v.at[cc + 2]], bufs[b], sems[b])

    return k(y, idx3)


FT = 1024
NF = F // FT


def _ffn_a_body(meta_ref, xs_ref, w1_ref, w3_ref, h_ref,
                w1c_ref, w3c_ref, key_ref):
    f = pl.program_id(0)
    b = pl.program_id(1)
    nvalid = meta_ref[NBLK]
    key = meta_ref[b] * NF + f

    @pl.when((key != key_ref[0]) | ((f == 0) & (b == 0)))
    def _():
        w1c_ref[...] = w1_ref[0].astype(jnp.bfloat16)
        w3c_ref[...] = w3_ref[0].astype(jnp.bfloat16)
        key_ref[0] = key

    @pl.when(b < nvalid)
    def _():
        xb = xs_ref[...].astype(jnp.bfloat16)
        gate = jnp.dot(xb, w1c_ref[...], preferred_element_type=jnp.float32)
        up = jnp.dot(xb, w3c_ref[...], preferred_element_type=jnp.float32)
        h_ref[...] = (gate * lax.logistic(gate) * up).astype(jnp.bfloat16)


def _ffn_b_body(meta_ref, h_ref, w2_ref, y_ref, w2c_ref, key_ref):
    d = pl.program_id(0)
    b = pl.program_id(1)
    nvalid = meta_ref[NBLK]
    key = meta_ref[b] * NF + d

    @pl.when((key != key_ref[0]) | ((d == 0) & (b == 0)))
    def _():
        w2c_ref[...] = w2_ref[0].astype(jnp.bfloat16)
        key_ref[0] = key

    @pl.when(b < nvalid)
    def _():
        y_ref[...] = jnp.dot(
            h_ref[...], w2c_ref[...], preferred_element_type=jnp.float32
        )


def _ffn(meta, xs, w1, w3, w2):
    def clamp(b, m):
        return jnp.minimum(b, m[NBLK] - 1)

    grid_a = pltpu.PrefetchScalarGridSpec(
        num_scalar_prefetch=1,
        grid=(NF, NBLK),
        in_specs=[
            pl.BlockSpec((BR, D), lambda f, b, m: (clamp(b, m), 0)),
            pl.BlockSpec((1, D, FT), lambda f, b, m: (m[b], 0, f)),
            pl.BlockSpec((1, D, FT), lambda f, b, m: (m[b], 0, f)),
        ],
        out_specs=pl.BlockSpec((BR, FT), lambda f, b, m: (b, f)),
        scratch_shapes=[
            pltpu.VMEM((D, FT), jnp.bfloat16),
            pltpu.VMEM((D, FT), jnp.bfloat16),
            pltpu.SMEM((1,), jnp.int32),
        ],
    )
    h = pl.pallas_call(
        _ffn_a_body,
        grid_spec=grid_a,
        out_shape=jax.ShapeDtypeStruct((NSLOT, F), jnp.bfloat16),
        compiler_params=pltpu.CompilerParams(
            dimension_semantics=("arbitrary", "arbitrary"),
        ),
    )(meta, xs, w1, w3)

    grid_b = pltpu.PrefetchScalarGridSpec(
        num_scalar_prefetch=1,
        grid=(NF, NBLK),
        in_specs=[
            pl.BlockSpec((BR, F), lambda d, b, m: (clamp(b, m), 0)),
            pl.BlockSpec((1, F, FT), lambda d, b, m: (m[b], 0, d)),
        ],
        out_specs=pl.BlockSpec((BR, FT), lambda d, b, m: (b, d)),
        scratch_shapes=[
            pltpu.VMEM((F, FT), jnp.bfloat16),
            pltpu.SMEM((1,), jnp.int32),
        ],
    )
    return pl.pallas_call(
        _ffn_b_body,
        grid_spec=grid_b,
        out_shape=jax.ShapeDtypeStruct((NSLOT, D), jnp.float32),
        compiler_params=pltpu.CompilerParams(
            dimension_semantics=("arbitrary", "arbitrary"),
        ),
    )(meta, h, w2)


def _combine_body(y0_ref, y1_ref, tw_ref, o_ref):
    tw = tw_ref[...]
    o_ref[...] = tw[:, 0:1] * y0_ref[...] + tw[:, 1:2] * y1_ref[...]


def _combine(yg, tw):
    nb = T // BR
    return pl.pallas_call(
        _combine_body,
        grid=(nb,),
        in_specs=[
            pl.BlockSpec((BR, D), lambda i: (i, 0)),
            pl.BlockSpec((BR, D), lambda i: (i + nb, 0)),
            pl.BlockSpec((BR, K), lambda i: (i, 0)),
        ],
        out_specs=pl.BlockSpec((BR, D), lambda i: (i, 0)),
        out_shape=jax.ShapeDtypeStruct((T, D), jnp.float32),
        compiler_params=pltpu.CompilerParams(
            dimension_semantics=("parallel",),
        ),
    )(yg, yg, tw)


@jax.jit
def kernel(x, router_logits, w1, w3, w2):
    tw, slots, meta = _route(router_logits)
    slots_flat = jnp.concatenate([slots[:, 0], slots[:, 1]])
    idx3 = jnp.reshape(slots_flat, (NW, NCHUNK, CHUNK))
    xs = _sc_scatter(x, idx3)
    y = _ffn(jnp.reshape(meta, (NBLK + 1,)), xs, w1, w3, w2)
    yg = _sc_gather(y, idx3)
    return _combine(yg, tw)


# speedup vs baseline: 1.7792x; 1.0208x over previous
import functools

import jax
import jax.numpy as jnp
from jax import lax
from jax.experimental import pallas as pl
from jax.experimental.pallas import tpu as pltpu
from jax.experimental.pallas import tpu_sc as plsc

T = 2048
D = 2048
F = 2048
E = 8
K = 2

BR = 256
NBLK = T * K // BR + E - 1
NSLOT = NBLK * BR

NC = 2
NS = 16
NW = NC * NS
CHUNK = 16
ROWS_PER_W = (T * K) // NW
NCHUNK = ROWS_PER_W // CHUNK


def _route_body(logits_ref, tw_ref, slots_ref, meta_ref):
    logits = logits_ref[...]
    m = jnp.max(logits, axis=-1, keepdims=True)
    ex = jnp.exp(logits - m)
    p = ex / jnp.sum(ex, axis=-1, keepdims=True)
    e_ids = lax.broadcasted_iota(jnp.int32, (T, E), 1)
    w0 = jnp.max(p, axis=-1, keepdims=True)
    i0 = jnp.min(jnp.where(p == w0, e_ids, E), axis=-1, keepdims=True)
    p2 = jnp.where(e_ids == i0, -jnp.inf, p)
    w1_ = jnp.max(p2, axis=-1, keepdims=True)
    i1 = jnp.min(jnp.where(p2 == w1_, e_ids, E), axis=-1, keepdims=True)
    sel0 = e_ids == i0
    sel1 = e_ids == i1
    cnt = (sel0 | sel1).astype(jnp.float32)
    ri = lax.broadcasted_iota(jnp.int32, (T, 1), 0)
    ci = lax.broadcasted_iota(jnp.int32, (1, T), 1)
    lt = (ci < ri).astype(jnp.bfloat16)
    rank = jnp.dot(lt, cnt.astype(jnp.bfloat16), preferred_element_type=jnp.float32)
    totals = jnp.sum(cnt, axis=0, keepdims=True)
    nblk = jnp.ceil(totals / BR)
    er = lax.broadcasted_iota(jnp.int32, (E, 1), 0)
    ec = lax.broadcasted_iota(jnp.int32, (1, E), 1)
    me = (er < ec).astype(jnp.float32)
    blk_off = jnp.dot(nblk, me, preferred_element_type=jnp.float32)
    slot = blk_off * BR + rank
    slot0 = jnp.sum(jnp.where(sel0, slot, 0.0), axis=-1, keepdims=True)
    slot1 = jnp.sum(jnp.where(sel1, slot, 0.0), axis=-1, keepdims=True)
    tw_ref[...] = jnp.concatenate([w0, w1_], axis=1)
    slots_ref[...] = jnp.concatenate([slot0, slot1], axis=1).astype(jnp.int32)
    b_ids = lax.broadcasted_iota(jnp.int32, (1, NBLK), 1)
    nb_i = jnp.sum(nblk).astype(jnp.int32)
    b_eff = jnp.minimum(b_ids, nb_i - 1).astype(jnp.float32)
    cmp = (b_eff >= jnp.reshape(blk_off, (E, 1))).astype(jnp.int32)
    be = jnp.sum(cmp, axis=0, keepdims=True) - 1
    meta_ref[...] = jnp.concatenate([be, jnp.reshape(nb_i, (1, 1))], axis=1)


def _route(router_logits):
    return pl.pallas_call(
        _route_body,
        out_shape=[
            jax.ShapeDtypeStruct((T, K), jnp.float32),
            jax.ShapeDtypeStruct((T, K), jnp.int32),
            jax.ShapeDtypeStruct((1, NBLK + 1), jnp.int32),
        ],
    )(router_logits)


def _sc_scatter(x, idx3):
    mesh = plsc.VectorSubcoreMesh(core_axis_name="c", subcore_axis_name="s")

    @functools.partial(
        pl.kernel,
        mesh=mesh,
        out_type=jax.ShapeDtypeStruct((NSLOT, D), jnp.float32),
        scratch_types=[
            pltpu.VMEM((NCHUNK, CHUNK), jnp.int32),
            pltpu.VMEM((CHUNK, D), jnp.float32),
            pltpu.VMEM((CHUNK, D), jnp.float32),
            pltpu.SemaphoreType.DMA,
            pltpu.SemaphoreType.DMA,
        ],
    )
    def k(x_hbm, idx_hbm, xs_hbm, idx_v, rows_a, rows_b, sem_a, sem_b):
        wid = lax.axis_index("s") * NC + lax.axis_index("c")
        tok0 = (wid % (T // ROWS_PER_W)) * ROWS_PER_W
        pltpu.sync_copy(idx_hbm.at[wid], idx_v)
        bufs = (rows_a, rows_b)
        sems = (sem_a, sem_b)
        for b in range(2):
            pltpu.async_copy(
                x_hbm.at[pl.ds(tok0 + b * CHUNK, CHUNK)], bufs[b], sems[b]
            )

        @pl.loop(0, NCHUNK // 2)
        def _(i):
            for b in range(2):
                cc = i * 2 + b
                pltpu.make_async_copy(
                    x_hbm.at[pl.ds(tok0, CHUNK)], bufs[b], sems[b]
                ).wait()
                pltpu.sync_copy(bufs[b], xs_hbm.at[idx_v.at[cc]])

                @pl.when(cc + 2 < NCHUNK)
                def _():
                    pltpu.async_copy(
                        x_hbm.at[pl.ds(tok0 + (cc + 2) * CHUNK, CHUNK)],
                        bufs[b],
                        sems[b],
                    )

    return k(x, idx3)


def _sc_gather(y, idx3):
    mesh = plsc.VectorSubcoreMesh(core_axis_name="c", subcore_axis_name="s")

    @functools.partial(
        pl.kernel,
        mesh=mesh,
        out_type=jax.ShapeDtypeStruct((T * K, D), jnp.float32),
        scratch_types=[
            pltpu.VMEM((NCHUNK, CHUNK), jnp.int32),
            pltpu.VMEM((CHUNK, D), jnp.float32),
            pltpu.VMEM((CHUNK, D), jnp.float32),
            pltpu.SemaphoreType.DMA,
            pltpu.SemaphoreType.DMA,
        ],
    )
    def k(y_hbm, idx_hbm, yg_hbm, idx_v, rows_a, rows_b, sem_a, sem_b):
        wid = lax.axis_index("s") * NC + lax.axis_index("c")
        base = wid * ROWS_PER_W
        pltpu.sync_copy(idx_hbm.at[wid], idx_v)
        bufs = (rows_a, rows_b)
        sems = (sem_a, sem_b)
        for b in range(2):
            pltpu.async_copy(y_hbm.at[idx_v.at[b]], bufs[b], sems[b])

        @pl.loop(0, NCHUNK // 2)
        def _(i):
            for b in range(2):
                cc = i * 2 + b
                pltpu.make_async_copy(
                    y_hbm.at[idx_v.at[0]], bufs[b], sems[b]
                ).wait()
                pltpu.sync_copy(
                    bufs[b], yg_hbm.at[pl.ds(base + cc * CHUNK, CHUNK)]
                )

                @pl.when(cc + 2 < NCHUNK)
                def _():
                    pltpu.async_copy(y_hbm.at[idx_v.at[cc + 2]], bufs[b], sems[b])

    return k(y, idx3)


FT = 1024
NF = F // FT


def _ffn(meta, xs, w1, w3, w2):
    tc_mesh = pltpu.create_tensorcore_mesh("core")

    @functools.partial(
        pl.kernel,
        mesh=tc_mesh,
        out_type=jax.ShapeDtypeStruct((NSLOT, F), jnp.bfloat16),
        scratch_types=[
            pltpu.SMEM((NBLK + 1,), jnp.int32),
            pltpu.VMEM((D, FT), jnp.bfloat16),
            pltpu.VMEM((D, FT), jnp.bfloat16),
            pltpu.SMEM((1,), jnp.int32),
            pltpu.SemaphoreType.DMA,
        ],
    )
    def ka(meta_hbm, xs_hbm, w1_hbm, w3_hbm, h_hbm, meta_s, w1c, w3c, keyc, sem):
        pltpu.async_copy(meta_hbm, meta_s, sem).wait()
        keyc[0] = -1
        nvalid = meta_s[NBLK]

        def body(idx, xs_b, w1_b, w3_b, h_b):
            f, b = idx
            key = meta_s[b] * NF + f

            @pl.when(key != keyc[0])
            def _():
                w1c[...] = w1_b[0].astype(jnp.bfloat16)
                w3c[...] = w3_b[0].astype(jnp.bfloat16)
                keyc[0] = key

            xb = xs_b[...].astype(jnp.bfloat16)
            gate = jnp.dot(xb, w1c[...], preferred_element_type=jnp.float32)
            up = jnp.dot(xb, w3c[...], preferred_element_type=jnp.float32)
            h_b[...] = (gate * lax.logistic(gate) * up).astype(jnp.bfloat16)

        pltpu.emit_pipeline(
            body,
            grid=(NF, nvalid),
            in_specs=[
                pl.BlockSpec((BR, D), lambda f, b: (b, 0)),
                pl.BlockSpec((1, D, FT), lambda f, b: (meta_s[b], 0, f)),
                pl.BlockSpec((1, D, FT), lambda f, b: (meta_s[b], 0, f)),
            ],
            out_specs=[pl.BlockSpec((BR, FT), lambda f, b: (b, f))],
            core_axis_name="core",
            dimension_semantics=(pltpu.PARALLEL, pltpu.ARBITRARY),
            _explicit_indices=True,
        )(xs_hbm, w1_hbm, w3_hbm, h_hbm)

    h = ka(meta, xs, w1, w3)

    @functools.partial(
        pl.kernel,
        mesh=tc_mesh,
        out_type=jax.ShapeDtypeStruct((NSLOT, D), jnp.float32),
        scratch_types=[
            pltpu.SMEM((NBLK + 1,), jnp.int32),
            pltpu.VMEM((F, FT), jnp.bfloat16),
            pltpu.SMEM((1,), jnp.int32),
            pltpu.SemaphoreType.DMA,
        ],
    )
    def kb(meta_hbm, h_hbm, w2_hbm, y_hbm, meta_s, w2c, keyc, sem):
        pltpu.async_copy(meta_hbm, meta_s, sem).wait()
        keyc[0] = -1
        nvalid = meta_s[NBLK]

        def body(idx, h_b, w2_b, y_b):
            d, b = idx
            key = meta_s[b] * NF + d

            @pl.when(key != keyc[0])
            def _():
                w2c[...] = w2_b[0].astype(jnp.bfloat16)
                keyc[0] = key

            y_b[...] = jnp.dot(
                h_b[...], w2c[...], preferred_element_type=jnp.float32
            )

        pltpu.emit_pipeline(
            body,
            grid=(NF, nvalid),
            in_specs=[
                pl.BlockSpec((BR, F), lambda d, b: (b, 0)),
                pl.BlockSpec((1, F, FT), lambda d, b: (meta_s[b], 0, d)),
            ],
            out_specs=[pl.BlockSpec((BR, FT), lambda d, b: (b, d))],
            core_axis_name="core",
            dimension_semantics=(pltpu.PARALLEL, pltpu.ARBITRARY),
            _explicit_indices=True,
        )(h_hbm, w2_hbm, y_hbm)

    return kb(meta, h, w2)


def _combine_body(y0_ref, y1_ref, tw_ref, o_ref):
    tw = tw_ref[...]
    o_ref[...] = tw[:, 0:1] * y0_ref[...] + tw[:, 1:2] * y1_ref[...]


def _combine(yg, tw):
    nb = T // BR
    return pl.pallas_call(
        _combine_body,
        grid=(nb,),
        in_specs=[
            pl.BlockSpec((BR, D), lambda i: (i, 0)),
            pl.BlockSpec((BR, D), lambda i: (i + nb, 0)),
            pl.BlockSpec((BR, K), lambda i: (i, 0)),
        ],
        out_specs=pl.BlockSpec((BR, D), lambda i: (i, 0)),
        out_shape=jax.ShapeDtypeStruct((T, D), jnp.float32),
        compiler_params=pltpu.CompilerParams(
            dimension_semantics=("parallel",),
        ),
    )(yg, yg, tw)


@jax.jit
def kernel(x, router_logits, w1, w3, w2):
    tw, slots, meta = _route(router_logits)
    slots_flat = jnp.concatenate([slots[:, 0], slots[:, 1]])
    idx3 = jnp.reshape(slots_flat, (NW, NCHUNK, CHUNK))
    xs = _sc_scatter(x, idx3)
    y = _ffn(jnp.reshape(meta, (NBLK + 1,)), xs, w1, w3, w2)
    yg = _sc_gather(y, idx3)
    return _combine(yg, tw)
